# Initial kernel scaffold; baseline (speedup 1.0000x reference)
import jax
import jax.numpy as jnp
from jax.experimental import pallas as pl


def kernel(Z, table):
    raise NotImplementedError("write your pallas kernel here")


# speedup vs baseline: 2.7666x; 2.7666x over previous
import functools

import jax
import jax.numpy as jnp
from jax import lax
from jax.experimental import pallas as pl
from jax.experimental.pallas import tpu as pltpu
from jax.experimental.pallas import tpu_sc as plsc

NUM_TYPES = 128
D_MODEL = 128
PADDING_IDX = 0

NC, NS = 2, 16
NW = NC * NS
G = 128
K = 8
C = G * K
HALF = C // 2


def _sc_body(idx_hbm, table_hbm, out_hbm, idx_v, rows_v, sem, *, steps, per_w):
    wid = lax.axis_index("c") * NS + lax.axis_index("s")

    def step(t, _):
        base = pl.multiple_of(wid * per_w + t * C, C)
        pltpu.sync_copy(idx_hbm.at[pl.ds(pl.multiple_of(base // G, K), K)], idx_v)
        for half in range(2):
            copies = [
                pltpu.async_copy(
                    table_hbm.at[idx_v.at[half * (K // 2) + j]],
                    rows_v.at[pl.ds(j * G, G)],
                    sem,
                )
                for j in range(K // 2)
            ]
            for cp in copies:
                cp.wait()
            pltpu.sync_copy(rows_v, out_hbm.at[pl.ds(base + half * HALF, HALF)])
        return ()

    lax.fori_loop(0, steps, step, (), unroll=False)


def kernel(Z, table):
    B = Z.size
    assert B % (NW * C) == 0, B
    assert table.shape == (NUM_TYPES, D_MODEL)
    per_w = B // NW
    steps = per_w // C

    idx = Z.reshape(B // G, G).astype(jnp.int32)
    table_eff = table.at[PADDING_IDX].set(0.0)

    mesh = plsc.VectorSubcoreMesh(core_axis_name="c", subcore_axis_name="s")
    run = pl.kernel(
        functools.partial(_sc_body, steps=steps, per_w=per_w),
        out_type=jax.ShapeDtypeStruct((B, D_MODEL), jnp.float32),
        mesh=mesh,
        scratch_types=[
            pltpu.VMEM((K, G), jnp.int32),
            pltpu.VMEM((HALF, D_MODEL), jnp.float32),
            pltpu.SemaphoreType.DMA,
        ],
    )
    out = run(idx, table_eff)
    return out.reshape(*Z.shape, D_MODEL)
